# baseline (speedup 1.0000x reference)
import functools

import jax
import jax.numpy as jnp
from jax import lax
from jax.experimental import pallas as pl
from jax.experimental.pallas import tpu as pltpu
from jax.experimental.pallas import tpu_sc as plsc

_N = 10000
_E = 160000
_D_IN = 256
_H = 512
_R = 16
_RD = 64
_T = 1024

_F32 = jnp.float32

_EB = 80
_NBATCH = _E // 16 // _EB
_EPT = _E // 16
_NP = 10240
_RPT = _NP // 16

_AB = 40
_ANB = _E // 32 // _AB


_AW = 128


def _aux_body(rid_ref, t_ref, wt1_ref, bt1_ref, out_ref):
    be = rid_ref.shape[0]
    r = rid_ref[...]
    lanes = lax.broadcasted_iota(jnp.int32, (be, _R), 1)
    onehot = (r == lanes).astype(_F32)
    tfp = jnp.maximum(t_ref[...] * wt1_ref[...] + bt1_ref[...], 0.0)
    pad = jnp.zeros((be, _AW - _R - _RD), _F32)
    out_ref[...] = jnp.concatenate([onehot, tfp, pad], axis=1)


def _build_aux(rel_ids, t, Wt1, bt1):
    be = 8000
    grid = _E // be
    return pl.pallas_call(
        _aux_body,
        grid=(grid,),
        in_specs=[
            pl.BlockSpec((be, 1), lambda i: (i, 0)),
            pl.BlockSpec((be, 1), lambda i: (i, 0)),
            pl.BlockSpec((1, _RD), lambda i: (0, 0)),
            pl.BlockSpec((1, _RD), lambda i: (0, 0)),
        ],
        out_specs=pl.BlockSpec((be, _AW), lambda i: (i, 0)),
        out_shape=jax.ShapeDtypeStruct((_E, _AW), _F32),
    )(rel_ids.reshape(_E, 1), t, Wt1.reshape(1, _RD), bt1.reshape(1, _RD))


def _h0_body(x_ref, w_ref, b_ref, out_ref):
    h = jnp.maximum(
        jnp.dot(x_ref[...], w_ref[...], preferred_element_type=_F32)
        + b_ref[...], 0.0)
    for c in range(4):
        out_ref[c] = h[:, c * 128:(c + 1) * 128]


def _input_proj(x, W_in, b_in):
    bn = 1024
    grid = _NP // bn
    return pl.pallas_call(
        _h0_body,
        grid=(grid,),
        in_specs=[
            pl.BlockSpec((bn, _D_IN), lambda i: (i, 0)),
            pl.BlockSpec((_D_IN, _H), lambda i: (0, 0)),
            pl.BlockSpec((1, _H), lambda i: (0, 0)),
        ],
        out_specs=pl.BlockSpec((4, bn, 128), lambda i: (0, i, 0)),
        out_shape=jax.ShapeDtypeStruct((4, _NP, 128), _F32),
    )(x, W_in, b_in.reshape(1, _H))


def _pre_body(h_ref, caux_ref, selfw_ref, rel_ref, wr_ref, wt_ref, wt2_ref,
              msgb_ref, bt2_ref, b2_ref, p_ref, u_ref):
    dot = functools.partial(jnp.dot, preferred_element_type=_F32)
    caux = caux_ref[0] + caux_ref[1]
    cnt = caux[:, :_R]
    atp = caux[:, _R:_R + _RD]
    deg = jnp.sum(cnt, axis=1, keepdims=True)
    wre = dot(rel_ref[...], wr_ref[...])
    wte = dot(wt2_ref[...], wt_ref[...])
    bvec = msgb_ref[...] + dot(bt2_ref[...], wt_ref[...])
    u_ref[...] = dot(cnt, wre) + dot(atp, wte) + deg * bvec
    p = b2_ref[...]
    for c in range(4):
        p = p + dot(h_ref[c], selfw_ref[c])
    p_ref[...] = p


def _layer_pre(h4, caux2, selfW, msgW, rel_emb, Wt2, msgb, bt2, selfb,
               neighb):
    bn = 1024
    grid = _NP // bn
    full = lambda shape: pl.BlockSpec(shape, lambda i: tuple(0 for _ in shape))
    return pl.pallas_call(
        _pre_body,
        grid=(grid,),
        in_specs=[
            pl.BlockSpec((4, bn, 128), lambda i: (0, i, 0)),
            pl.BlockSpec((2, bn, _AW), lambda i: (0, i, 0)),
            full((4, 128, _H)),
            full((_R, _RD)),
            full((_RD, _H)),
            full((_RD, _H)),
            full((_RD, _RD)),
            full((1, _H)),
            full((1, _RD)),
            full((1, _H)),
        ],
        out_specs=[
            pl.BlockSpec((bn, _H), lambda i: (i, 0)),
            pl.BlockSpec((bn, _H), lambda i: (i, 0)),
        ],
        out_shape=[
            jax.ShapeDtypeStruct((_NP, _H), _F32),
            jax.ShapeDtypeStruct((_NP, _H), _F32),
        ],
    )(h4, caux2, selfW.reshape(4, 128, _H), rel_emb, msgW[_H:_H + _RD],
      msgW[_H + _RD:], Wt2, msgb.reshape(1, _H), bt2.reshape(1, _RD),
      (selfb + neighb).reshape(1, _H))


def _post_body(p_ref, u_ref, caux_ref, g_ref, wh_ref, neighw_ref, out_ref):
    dot = functools.partial(jnp.dot, preferred_element_type=_F32)
    caux = caux_ref[0] + caux_ref[1]
    deg = jnp.sum(caux[:, :_R], axis=1, keepdims=True)
    u = u_ref[...]
    for c in range(4):
        u += dot(g_ref[c], wh_ref[c])
    agg = u / jnp.maximum(deg, 1.0)
    hn = jnp.maximum(p_ref[...] + dot(agg, neighw_ref[...]), 0.0)
    for c in range(4):
        out_ref[c] = hn[:, c * 128:(c + 1) * 128]


def _layer_post(p, u, caux2, g4, msgW, neighW):
    bn = 1024
    grid = _NP // bn
    full = lambda shape: pl.BlockSpec(shape, lambda i: tuple(0 for _ in shape))
    return pl.pallas_call(
        _post_body,
        grid=(grid,),
        in_specs=[
            pl.BlockSpec((bn, _H), lambda i: (i, 0)),
            pl.BlockSpec((bn, _H), lambda i: (i, 0)),
            pl.BlockSpec((2, bn, _AW), lambda i: (0, i, 0)),
            pl.BlockSpec((4, bn, 128), lambda i: (0, i, 0)),
            full((4, 128, _H)),
            full((_H, _H)),
        ],
        out_specs=pl.BlockSpec((4, bn, 128), lambda i: (0, i, 0)),
        out_shape=jax.ShapeDtypeStruct((4, _NP, 128), _F32),
    )(p, u, caux2, g4, msgW[:_H].reshape(4, 128, _H), neighW)


def _head_body(ht_ref, wc1_ref, bc1_ref, wc2_ref, bc2_ref, out_ref):
    dot = functools.partial(jnp.dot, preferred_element_type=_F32)
    s = bc1_ref[...]
    for c in range(4):
        s = s + dot(ht_ref[c], wc1_ref[c])
    a = jnp.maximum(s, 0.0)
    out_ref[...] = dot(a, wc2_ref[...]) + bc2_ref[...]


def _head(ht4, Wc1, bc1, Wc2, bc2):
    full = lambda shape: pl.BlockSpec(shape, lambda: tuple(0 for _ in shape))
    return pl.pallas_call(
        _head_body,
        in_specs=[
            full((4, _T, 128)),
            full((4, 128, _H)),
            full((1, _H)),
            full((_H, 1)),
            full((1, 1)),
        ],
        out_specs=full((_T, 1)),
        out_shape=jax.ShapeDtypeStruct((_T, 1), _F32),
    )(ht4, Wc1.reshape(4, 128, _H), bc1.reshape(1, _H), Wc2,
      bc2.reshape(1, 1))


@functools.cache
def _mesh():
    return plsc.VectorSubcoreMesh(core_axis_name="c", subcore_axis_name="s")


def _aux_scatter_kernel(aux_hbm, dst_hbm, zeros_hbm, caux_hbm,
                        acc, dst_v, buf_a, buf_b, buf_c,
                        sem_a, sem_b, sem_c):
    c = lax.axis_index("c")
    s = lax.axis_index("s")
    w = c * 16 + s
    bufs = (buf_a, buf_b, buf_c)
    sems = (sem_a, sem_b, sem_c)
    pltpu.sync_copy(zeros_hbm.at[pl.ds(s * _RPT, _RPT)],
                    acc.at[pl.ds(s * _RPT, _RPT)])
    pltpu.sync_copy(dst_hbm.at[w, pl.ds(0, _DRING)], dst_v)
    plsc.subcore_barrier()

    ebase = w * (_ANB * _AB)

    def aslice(j):
        return aux_hbm.at[pl.ds(ebase + j * _AB, _AB)]

    for k in range(3):
        pltpu.async_copy(aslice(k), bufs[k], sems[k])

    def step(j, k, reload=True):
        jm = lax.rem(j, _DRING)
        if reload:
            @pl.when(jnp.logical_and(jm == 0, j > 0))
            def _():
                pltpu.sync_copy(
                    dst_hbm.at[w, pl.ds(pl.multiple_of(j, _DRING), _DRING)],
                    dst_v)
        pltpu.make_async_copy(aslice(j), bufs[k], sems[k]).wait()
        pltpu.sync_copy(bufs[k], acc.at[dst_v.at[jm]], add=True)

        @pl.when(j + 3 < _ANB)
        def _():
            pltpu.async_copy(aslice(j + 3), bufs[k], sems[k])

    def body(i, carry):
        for k in range(3):
            step(i * 3 + k, k)
        return carry

    nloop = _ANB // 3
    lax.fori_loop(0, nloop, body, 0)
    for k in range(_ANB - nloop * 3):
        step(nloop * 3 + k, k, reload=False)
    plsc.subcore_barrier()
    pltpu.sync_copy(
        acc.at[pl.ds(s * _RPT, _RPT)],
        caux_hbm.at[pl.ds(c * _NP + s * _RPT, _RPT)])


def _aux_scatter(aux, dstA):
    zeros = jnp.zeros((_NP, _AW), _F32)
    k = pl.kernel(
        _aux_scatter_kernel,
        out_type=jax.ShapeDtypeStruct((2 * _NP, _AW), _F32),
        mesh=_mesh(),
        scratch_types=[
            pltpu.VMEM_SHARED((_NP, _AW), _F32),
            pltpu.VMEM((_DRING, _AB), jnp.int32),
            pltpu.VMEM((_AB, _AW), _F32),
            pltpu.VMEM((_AB, _AW), _F32),
            pltpu.VMEM((_AB, _AW), _F32),
            pltpu.SemaphoreType.DMA,
            pltpu.SemaphoreType.DMA,
            pltpu.SemaphoreType.DMA,
        ],
    )
    return k(aux, dstA, zeros).reshape(2, _NP, _AW)


_DRING = 32


def _g_scatter_kernel(h_hbm, src_hbm, dst_hbm, zeros_hbm, g_hbm,
                      acc, src_v, dst_v, buf_a, buf_b, buf_c,
                      sem_a, sem_b, sem_c):
    c = lax.axis_index("c")
    s = lax.axis_index("s")
    pltpu.sync_copy(src_hbm.at[pl.ds(s * _EPT, _EPT)], src_v)
    bufs = (buf_a, buf_b, buf_c)
    sems = (sem_a, sem_b, sem_c)
    for p in range(2):
        chunk = c * 2 + p
        tbl = h_hbm.at[pl.ds(chunk * _NP, _NP)]
        pltpu.sync_copy(zeros_hbm.at[pl.ds(s * _RPT, _RPT)],
                        acc.at[pl.ds(s * _RPT, _RPT)])
        plsc.subcore_barrier()

        def sidx(j):
            return src_v.at[pl.ds(j * _EB, _EB)]

        pltpu.sync_copy(dst_hbm.at[s, pl.ds(0, _DRING)], dst_v)
        for k in range(3):
            pltpu.async_copy(tbl.at[sidx(k)], bufs[k], sems[k])

        def step(j, k, reload=True):
            jm = lax.rem(j, _DRING)

            if reload:
                @pl.when(jnp.logical_and(jm == 0, j > 0))
                def _():
                    pltpu.sync_copy(
                        dst_hbm.at[s, pl.ds(pl.multiple_of(j, _DRING),
                                            _DRING)], dst_v)

            pltpu.make_async_copy(tbl.at[sidx(j)], bufs[k], sems[k]).wait()
            pltpu.sync_copy(bufs[k], acc.at[dst_v.at[jm]], add=True)

            @pl.when(j + 3 < _NBATCH)
            def _():
                pltpu.async_copy(tbl.at[sidx(j + 3)], bufs[k], sems[k])

        def body(i, carry):
            for k in range(3):
                step(i * 3 + k, k)
            return carry

        nloop = _NBATCH // 3
        lax.fori_loop(0, nloop, body, 0)
        for k in range(_NBATCH - nloop * 3):
            step(nloop * 3 + k, k, reload=False)
        plsc.subcore_barrier()
        pltpu.sync_copy(
            acc.at[pl.ds(s * _RPT, _RPT)],
            g_hbm.at[pl.ds(chunk * _NP + s * _RPT, _RPT)])
        plsc.subcore_barrier()


def _g_scatter(h4, src, dstP):
    zeros = jnp.zeros((_NP, 128), _F32)
    k = pl.kernel(
        _g_scatter_kernel,
        out_type=jax.ShapeDtypeStruct((4 * _NP, 128), _F32),
        mesh=_mesh(),
        scratch_types=[
            pltpu.VMEM_SHARED((_NP, 128), _F32),
            pltpu.VMEM((_EPT,), jnp.int32),
            pltpu.VMEM((_DRING, _EB), jnp.int32),
            pltpu.VMEM((_EB, 128), _F32),
            pltpu.VMEM((_EB, 128), _F32),
            pltpu.VMEM((_EB, 128), _F32),
            pltpu.SemaphoreType.DMA,
            pltpu.SemaphoreType.DMA,
            pltpu.SemaphoreType.DMA,
        ],
    )
    return k(h4.reshape(4 * _NP, 128), src, dstP, zeros).reshape(
        4, _NP, 128)


def _tgt_gather_kernel(h_hbm, idx_hbm, out_hbm, idx_v, buf, sem):
    c = lax.axis_index("c")
    s = lax.axis_index("s")
    w = c * 16 + s
    pltpu.sync_copy(idx_hbm.at[pl.ds(w * 128, 128)], idx_v)
    pltpu.async_copy(h_hbm.at[idx_v], buf, sem).wait()
    pltpu.sync_copy(buf, out_hbm.at[pl.ds(w * 128, 128)])


def _tgt_gather(h4, tgtoff):
    k = pl.kernel(
        _tgt_gather_kernel,
        out_type=jax.ShapeDtypeStruct((4 * _T, 128), _F32),
        mesh=_mesh(),
        scratch_types=[
            pltpu.VMEM((128,), jnp.int32),
            pltpu.VMEM((128, 128), _F32),
            pltpu.SemaphoreType.DMA,
        ],
    )
    return k(h4.reshape(4 * _NP, 128), tgtoff).reshape(4, _T, 128)


def kernel(x, edge_relative_time, W_in, b_in, Wt1, bt1, Wt2, bt2,
           rel_emb0, msgW0, msgb0, selfW0, selfb0, neighW0, neighb0,
           rel_emb1, msgW1, msgb1, selfW1, selfb1, neighW1, neighb1,
           Wc1, bc1, Wc2, bc2,
           edge_src, edge_dst, rel_ids, target_local_idx):
    src = edge_src.astype(jnp.int32)
    dst = edge_dst.astype(jnp.int32)
    rid = rel_ids.astype(jnp.int32)
    tgt = target_local_idx.astype(jnp.int32)

    chunk_off = (jnp.arange(4, dtype=jnp.int32) * _NP)[:, None]
    dstP = jnp.pad(dst.reshape(16, _NBATCH, _EB),
                   ((0, 0), (0, 128 - _NBATCH), (0, 0)))
    dstA = jnp.pad(dst.reshape(32, _ANB, _AB),
                   ((0, 0), (0, 128 - _ANB), (0, 0)))
    tgtoff = (tgt.reshape(1, _T) + chunk_off).reshape(4 * _T)

    aux = _build_aux(rid, edge_relative_time, Wt1, bt1)
    caux2 = _aux_scatter(aux, dstA)
    h = _input_proj(x, W_in, b_in)

    g = _g_scatter(h, src, dstP)
    p, u = _layer_pre(h, caux2, selfW0, msgW0, rel_emb0, Wt2, msgb0, bt2,
                      selfb0, neighb0)
    h = _layer_post(p, u, caux2, g, msgW0, neighW0)
    g = _g_scatter(h, src, dstP)
    p, u = _layer_pre(h, caux2, selfW1, msgW1, rel_emb1, Wt2, msgb1, bt2,
                      selfb1, neighb1)
    h = _layer_post(p, u, caux2, g, msgW1, neighW1)

    ht = _tgt_gather(h, tgtoff)
    out = _head(ht, Wc1, bc1, Wc2, bc2)
    return out[:, 0]


# speedup vs baseline: 1.1776x; 1.1776x over previous
import functools

import jax
import jax.numpy as jnp
from jax import lax
from jax.experimental import pallas as pl
from jax.experimental.pallas import tpu as pltpu
from jax.experimental.pallas import tpu_sc as plsc

_N = 10000
_E = 160000
_D_IN = 256
_H = 512
_R = 16
_RD = 64
_T = 1024

_F32 = jnp.float32

_EB = 80
_NBATCH = _E // 16 // _EB
_EPT = _E // 16
_NP = 10240
_RPT = _NP // 16

_AB = 40
_ANB = _E // 32 // _AB


_AW = 128


def _aux_body(rid_ref, t_ref, wt1_ref, bt1_ref, out_ref):
    be = rid_ref.shape[0]
    r = rid_ref[...]
    lanes = lax.broadcasted_iota(jnp.int32, (be, _R), 1)
    onehot = (r == lanes).astype(_F32)
    tfp = jnp.maximum(t_ref[...] * wt1_ref[...] + bt1_ref[...], 0.0)
    pad = jnp.zeros((be, _AW - _R - _RD), _F32)
    out_ref[...] = jnp.concatenate([onehot, tfp, pad], axis=1)


def _build_aux(rel_ids, t, Wt1, bt1):
    be = 8000
    grid = _E // be
    return pl.pallas_call(
        _aux_body,
        grid=(grid,),
        in_specs=[
            pl.BlockSpec((be, 1), lambda i: (i, 0)),
            pl.BlockSpec((be, 1), lambda i: (i, 0)),
            pl.BlockSpec((1, _RD), lambda i: (0, 0)),
            pl.BlockSpec((1, _RD), lambda i: (0, 0)),
        ],
        out_specs=pl.BlockSpec((be, _AW), lambda i: (i, 0)),
        out_shape=jax.ShapeDtypeStruct((_E, _AW), _F32),
    )(rel_ids.reshape(_E, 1), t, Wt1.reshape(1, _RD), bt1.reshape(1, _RD))


def _h0_body(x_ref, w_ref, b_ref, out_ref):
    h = jnp.maximum(
        jnp.dot(x_ref[...], w_ref[...], preferred_element_type=_F32)
        + b_ref[...], 0.0)
    for c in range(4):
        out_ref[c] = h[:, c * 128:(c + 1) * 128]


def _input_proj(x, W_in, b_in):
    bn = 1024
    grid = _NP // bn
    return pl.pallas_call(
        _h0_body,
        grid=(grid,),
        in_specs=[
            pl.BlockSpec((bn, _D_IN), lambda i: (i, 0)),
            pl.BlockSpec((_D_IN, _H), lambda i: (0, 0)),
            pl.BlockSpec((1, _H), lambda i: (0, 0)),
        ],
        out_specs=pl.BlockSpec((4, bn, 128), lambda i: (0, i, 0)),
        out_shape=jax.ShapeDtypeStruct((4, _NP, 128), _F32),
    )(x, W_in, b_in.reshape(1, _H))


def _layer_body(h_ref, g_ref, caux_ref, selfw_ref, wh_ref, neighw_ref,
                rel_ref, wr_ref, wt_ref, wt2_ref, msgb_ref, bt2_ref, b2_ref,
                out_ref):
    dot = functools.partial(jnp.dot, preferred_element_type=_F32)
    caux = caux_ref[0] + caux_ref[1]
    cnt = caux[:, :_R]
    atp = caux[:, _R:_R + _RD]
    deg = jnp.sum(cnt, axis=1, keepdims=True)
    wre = dot(rel_ref[...], wr_ref[...])
    wte = dot(wt2_ref[...], wt_ref[...])
    bvec = msgb_ref[...] + dot(bt2_ref[...], wt_ref[...])
    u = dot(cnt, wre) + dot(atp, wte) + deg * bvec
    for c in range(4):
        u += dot(g_ref[c], wh_ref[c])
    agg = u / jnp.maximum(deg, 1.0)
    s = dot(agg, neighw_ref[...]) + b2_ref[...]
    for c in range(4):
        s += dot(h_ref[c], selfw_ref[c])
    hn = jnp.maximum(s, 0.0)
    for c in range(4):
        out_ref[c] = hn[:, c * 128:(c + 1) * 128]


def _layer_combine(h4, g4, caux2, selfW, msgW, neighW, rel_emb, Wt2, msgb,
                   bt2, selfb, neighb):
    bn = 1024
    grid = _NP // bn
    full = lambda shape: pl.BlockSpec(shape, lambda i: tuple(0 for _ in shape))
    selfw4 = selfW.reshape(4, 128, _H)
    wh4 = msgW[:_H].reshape(4, 128, _H)
    wr = msgW[_H:_H + _RD]
    wt = msgW[_H + _RD:]
    b2 = (selfb + neighb).reshape(1, _H)
    return pl.pallas_call(
        _layer_body,
        grid=(grid,),
        in_specs=[
            pl.BlockSpec((4, bn, 128), lambda i: (0, i, 0)),
            pl.BlockSpec((4, bn, 128), lambda i: (0, i, 0)),
            pl.BlockSpec((2, bn, _AW), lambda i: (0, i, 0)),
            full((4, 128, _H)),
            full((4, 128, _H)),
            full((_H, _H)),
            full((_R, _RD)),
            full((_RD, _H)),
            full((_RD, _H)),
            full((_RD, _RD)),
            full((1, _H)),
            full((1, _RD)),
            full((1, _H)),
        ],
        out_specs=pl.BlockSpec((4, bn, 128), lambda i: (0, i, 0)),
        out_shape=jax.ShapeDtypeStruct((4, _NP, 128), _F32),
    )(h4, g4, caux2, selfw4, wh4, neighW, rel_emb, wr, wt, Wt2,
      msgb.reshape(1, _H), bt2.reshape(1, _RD), b2)


def _head_body(ht_ref, wc1_ref, bc1_ref, wc2_ref, bc2_ref, out_ref):
    dot = functools.partial(jnp.dot, preferred_element_type=_F32)
    s = bc1_ref[...]
    for c in range(4):
        s = s + dot(ht_ref[c], wc1_ref[c])
    a = jnp.maximum(s, 0.0)
    out_ref[...] = dot(a, wc2_ref[...]) + bc2_ref[...]


def _head(ht4, Wc1, bc1, Wc2, bc2):
    full = lambda shape: pl.BlockSpec(shape, lambda: tuple(0 for _ in shape))
    return pl.pallas_call(
        _head_body,
        in_specs=[
            full((4, _T, 128)),
            full((4, 128, _H)),
            full((1, _H)),
            full((_H, 1)),
            full((1, 1)),
        ],
        out_specs=full((_T, 1)),
        out_shape=jax.ShapeDtypeStruct((_T, 1), _F32),
    )(ht4, Wc1.reshape(4, 128, _H), bc1.reshape(1, _H), Wc2,
      bc2.reshape(1, 1))


@functools.cache
def _mesh():
    return plsc.VectorSubcoreMesh(core_axis_name="c", subcore_axis_name="s")


def _aux_scatter_kernel(aux_hbm, dst_hbm, zeros_hbm, caux_hbm,
                        acc, dst_v, buf_a, buf_b, buf_c,
                        sem_a, sem_b, sem_c):
    c = lax.axis_index("c")
    s = lax.axis_index("s")
    w = c * 16 + s
    bufs = (buf_a, buf_b, buf_c)
    sems = (sem_a, sem_b, sem_c)
    pltpu.sync_copy(zeros_hbm.at[pl.ds(s * _RPT, _RPT)],
                    acc.at[pl.ds(s * _RPT, _RPT)])
    pltpu.sync_copy(dst_hbm.at[w, pl.ds(0, _DRING)], dst_v)
    plsc.subcore_barrier()

    ebase = w * (_ANB * _AB)

    def aslice(j):
        return aux_hbm.at[pl.ds(ebase + j * _AB, _AB)]

    for k in range(3):
        pltpu.async_copy(aslice(k), bufs[k], sems[k])

    def step(j, k, reload=True):
        jm = lax.rem(j, _DRING)
        if reload:
            @pl.when(jnp.logical_and(jm == 0, j > 0))
            def _():
                pltpu.sync_copy(
                    dst_hbm.at[w, pl.ds(pl.multiple_of(j, _DRING), _DRING)],
                    dst_v)
        pltpu.make_async_copy(aslice(j), bufs[k], sems[k]).wait()
        pltpu.sync_copy(bufs[k], acc.at[dst_v.at[jm]], add=True)

        @pl.when(j + 3 < _ANB)
        def _():
            pltpu.async_copy(aslice(j + 3), bufs[k], sems[k])

    def body(i, carry):
        for k in range(3):
            step(i * 3 + k, k)
        return carry

    nloop = _ANB // 3
    lax.fori_loop(0, nloop, body, 0)
    for k in range(_ANB - nloop * 3):
        step(nloop * 3 + k, k, reload=False)
    plsc.subcore_barrier()
    pltpu.sync_copy(
        acc.at[pl.ds(s * _RPT, _RPT)],
        caux_hbm.at[pl.ds(c * _NP + s * _RPT, _RPT)])


def _aux_scatter(aux, dstA):
    zeros = jnp.zeros((_NP, _AW), _F32)
    k = pl.kernel(
        _aux_scatter_kernel,
        out_type=jax.ShapeDtypeStruct((2 * _NP, _AW), _F32),
        mesh=_mesh(),
        scratch_types=[
            pltpu.VMEM_SHARED((_NP, _AW), _F32),
            pltpu.VMEM((_DRING, _AB), jnp.int32),
            pltpu.VMEM((_AB, _AW), _F32),
            pltpu.VMEM((_AB, _AW), _F32),
            pltpu.VMEM((_AB, _AW), _F32),
            pltpu.SemaphoreType.DMA,
            pltpu.SemaphoreType.DMA,
            pltpu.SemaphoreType.DMA,
        ],
    )
    return k(aux, dstA, zeros).reshape(2, _NP, _AW)


_DRING = 32


def _g_scatter_kernel(h_hbm, src_hbm, dst_hbm, zeros_hbm, g_hbm,
                      acc, src_v, dst_v, buf_a, buf_b, buf_c,
                      sem_a, sem_b, sem_c):
    c = lax.axis_index("c")
    s = lax.axis_index("s")
    pltpu.sync_copy(src_hbm.at[pl.ds(s * _EPT, _EPT)], src_v)
    bufs = (buf_a, buf_b, buf_c)
    sems = (sem_a, sem_b, sem_c)
    for p in range(2):
        chunk = c * 2 + p
        tbl = h_hbm.at[pl.ds(chunk * _NP, _NP)]
        pltpu.sync_copy(zeros_hbm.at[pl.ds(s * _RPT, _RPT)],
                        acc.at[pl.ds(s * _RPT, _RPT)])
        plsc.subcore_barrier()

        def sidx(j):
            return src_v.at[pl.ds(j * _EB, _EB)]

        pltpu.sync_copy(dst_hbm.at[s, pl.ds(0, _DRING)], dst_v)
        for k in range(3):
            pltpu.async_copy(tbl.at[sidx(k)], bufs[k], sems[k])

        def step(j, k, reload=True):
            jm = lax.rem(j, _DRING)

            if reload:
                @pl.when(jnp.logical_and(jm == 0, j > 0))
                def _():
                    pltpu.sync_copy(
                        dst_hbm.at[s, pl.ds(pl.multiple_of(j, _DRING),
                                            _DRING)], dst_v)

            pltpu.make_async_copy(tbl.at[sidx(j)], bufs[k], sems[k]).wait()
            pltpu.sync_copy(bufs[k], acc.at[dst_v.at[jm]], add=True)

            @pl.when(j + 3 < _NBATCH)
            def _():
                pltpu.async_copy(tbl.at[sidx(j + 3)], bufs[k], sems[k])

        def body(i, carry):
            for k in range(3):
                step(i * 3 + k, k)
            return carry

        nloop = _NBATCH // 3
        lax.fori_loop(0, nloop, body, 0)
        for k in range(_NBATCH - nloop * 3):
            step(nloop * 3 + k, k, reload=False)
        plsc.subcore_barrier()
        pltpu.sync_copy(
            acc.at[pl.ds(s * _RPT, _RPT)],
            g_hbm.at[pl.ds(chunk * _NP + s * _RPT, _RPT)])
        plsc.subcore_barrier()


@functools.cache
def _g_kernel():
    return pl.kernel(
        _g_scatter_kernel,
        out_type=jax.ShapeDtypeStruct((4 * _NP, 128), _F32),
        mesh=_mesh(),
        scratch_types=[
            pltpu.VMEM_SHARED((_NP, 128), _F32),
            pltpu.VMEM((_EPT,), jnp.int32),
            pltpu.VMEM((_DRING, _EB), jnp.int32),
            pltpu.VMEM((_EB, 128), _F32),
            pltpu.VMEM((_EB, 128), _F32),
            pltpu.VMEM((_EB, 128), _F32),
            pltpu.SemaphoreType.DMA,
            pltpu.SemaphoreType.DMA,
            pltpu.SemaphoreType.DMA,
        ],
    )


def _g_scatter(h4, src, dstP):
    zeros = jnp.zeros((_NP, 128), _F32)
    return _g_kernel()(h4.reshape(4 * _NP, 128), src, dstP, zeros).reshape(
        4, _NP, 128)


def _tgt_gather_kernel(h_hbm, idx_hbm, out_hbm, idx_v, buf, sem):
    c = lax.axis_index("c")
    s = lax.axis_index("s")
    w = c * 16 + s
    pltpu.sync_copy(idx_hbm.at[pl.ds(w * 128, 128)], idx_v)
    pltpu.async_copy(h_hbm.at[idx_v], buf, sem).wait()
    pltpu.sync_copy(buf, out_hbm.at[pl.ds(w * 128, 128)])


def _tgt_gather(h4, tgtoff):
    k = pl.kernel(
        _tgt_gather_kernel,
        out_type=jax.ShapeDtypeStruct((4 * _T, 128), _F32),
        mesh=_mesh(),
        scratch_types=[
            pltpu.VMEM((128,), jnp.int32),
            pltpu.VMEM((128, 128), _F32),
            pltpu.SemaphoreType.DMA,
        ],
    )
    return k(h4.reshape(4 * _NP, 128), tgtoff).reshape(4, _T, 128)


def kernel(x, edge_relative_time, W_in, b_in, Wt1, bt1, Wt2, bt2,
           rel_emb0, msgW0, msgb0, selfW0, selfb0, neighW0, neighb0,
           rel_emb1, msgW1, msgb1, selfW1, selfb1, neighW1, neighb1,
           Wc1, bc1, Wc2, bc2,
           edge_src, edge_dst, rel_ids, target_local_idx):
    src = edge_src.astype(jnp.int32)
    dst = edge_dst.astype(jnp.int32)
    rid = rel_ids.astype(jnp.int32)
    tgt = target_local_idx.astype(jnp.int32)

    chunk_off = (jnp.arange(4, dtype=jnp.int32) * _NP)[:, None]
    dstP = jnp.pad(dst.reshape(16, _NBATCH, _EB),
                   ((0, 0), (0, 128 - _NBATCH), (0, 0)))
    dstA = jnp.pad(dst.reshape(32, _ANB, _AB),
                   ((0, 0), (0, 128 - _ANB), (0, 0)))
    tgtoff = (tgt.reshape(1, _T) + chunk_off).reshape(4 * _T)

    aux = _build_aux(rid, edge_relative_time, Wt1, bt1)
    caux2 = _aux_scatter(aux, dstA)
    h = _input_proj(x, W_in, b_in)

    g = _g_scatter(h, src, dstP)
    h = _layer_combine(h, g, caux2, selfW0, msgW0, neighW0, rel_emb0,
                       Wt2, msgb0, bt2, selfb0, neighb0)
    g = _g_scatter(h, src, dstP)
    h = _layer_combine(h, g, caux2, selfW1, msgW1, neighW1, rel_emb1,
                       Wt2, msgb1, bt2, selfb1, neighb1)

    ht = _tgt_gather(h, tgtoff)
    out = _head(ht, Wc1, bc1, Wc2, bc2)
    return out[:, 0]
